# baseline (speedup 1.0000x reference)
import functools

import jax
import jax.numpy as jnp
from jax import lax
from jax.experimental import pallas as pl
from jax.experimental.pallas import tpu as pltpu
from jax.experimental.pallas import tpu_sc as plsc

NC = 2
NS = 16
NW = NC * NS

N_PAD = 10112
ROWS_PER_TILE = N_PAD // NS
CHUNK = 128
NB_CORE0 = 20
NB_CORE1 = 60
FW = 128
CW = 16


def _make_sc_aggregate(nb0, nb1, nc_max):
  mesh = plsc.VectorSubcoreMesh(core_axis_name="c", subcore_axis_name="s")

  def body(table, src3, dst3, zrow, sums_out,
           acc, src_v, dst_v, buf, gs0, gs1, ss0, ss1):
    c = lax.axis_index("c")
    s = lax.axis_index("s")
    wid = c * NS + s
    nb = jnp.where(c == 0, nb0, nb1)
    base = s * ROWS_PER_TILE
    pltpu.sync_copy(src3.at[wid], src_v)
    pltpu.sync_copy(dst3.at[wid], dst_v)
    pltpu.sync_copy(zrow, acc.at[pl.ds(base, ROWS_PER_TILE)])
    plsc.subcore_barrier()

    gsem = (gs0, gs1)
    ssem = (ss0, ss1)

    def start_gather(j, b):
      return pltpu.async_copy(table.at[src_v.at[j]], buf.at[b], gsem[b])

    def wait_gather(b):
      pltpu.make_async_copy(table.at[src_v.at[0]], buf.at[b],
                            gsem[b]).wait()

    def start_scatter(j, b):
      return pltpu.async_copy(buf.at[b], acc.at[dst_v.at[j]], ssem[b],
                              add=True)

    def wait_scatter(b):
      pltpu.make_async_copy(buf.at[b], acc.at[dst_v.at[0]], ssem[b]).wait()

    start_gather(0, 0)
    start_gather(1, 1)

    def loop(i, carry):
      g = 2 * i
      wait_gather(0)
      start_scatter(g, 0)
      wait_gather(1)
      start_scatter(g + 1, 1)
      wait_scatter(0)
      start_gather(g + 2, 0)
      wait_scatter(1)
      start_gather(g + 3, 1)
      return carry

    lax.fori_loop(0, nb // 2 - 1, loop, 0)
    wait_gather(0)
    d0 = start_scatter(nb - 2, 0)
    wait_gather(1)
    d1 = start_scatter(nb - 1, 1)
    d0.wait()
    d1.wait()
    plsc.subcore_barrier()
    pltpu.sync_copy(acc.at[pl.ds(base, ROWS_PER_TILE)],
                    sums_out.at[c, pl.ds(base, ROWS_PER_TILE)])

  return pl.kernel(
      body,
      out_type=[jax.ShapeDtypeStruct((NC, N_PAD, FW), jnp.float32)],
      mesh=mesh,
      scratch_types=[
          pltpu.VMEM_SHARED((N_PAD, FW), jnp.float32),
          pltpu.VMEM((nc_max, CHUNK), jnp.int32),
          pltpu.VMEM((nc_max, CHUNK), jnp.int32),
          pltpu.VMEM((2, CHUNK, FW), jnp.float32),
          pltpu.SemaphoreType.DMA,
          pltpu.SemaphoreType.DMA,
          pltpu.SemaphoreType.DMA,
          pltpu.SemaphoreType.DMA,
      ])


CROWS = N_PAD // 128


def _make_sc_degree(n_chunks):
  mesh = plsc.VectorSubcoreMesh(core_axis_name="c", subcore_axis_name="s")

  def body(dst3, zrow, ones, cnts_out, cacc, dst_v, ones_v):
    c = lax.axis_index("c")
    s = lax.axis_index("s")
    wid = s * NC + c
    base = s * ROWS_PER_TILE
    pltpu.sync_copy(dst3.at[wid], dst_v)
    pltpu.sync_copy(ones, ones_v)
    pltpu.sync_copy(zrow, cacc.at[pl.ds(base, ROWS_PER_TILE)])
    plsc.subcore_barrier()

    def loop(j, carry):
      pltpu.sync_copy(ones_v, cacc.at[dst_v.at[j]], add=True)
      return carry

    lax.fori_loop(0, n_chunks, loop, 0)
    plsc.subcore_barrier()
    pltpu.sync_copy(cacc.at[pl.ds(base, ROWS_PER_TILE)],
                    cnts_out.at[c, pl.ds(base, ROWS_PER_TILE)])

  return pl.kernel(
      body,
      out_type=[jax.ShapeDtypeStruct((NC, N_PAD, FW), jnp.float32)],
      mesh=mesh,
      scratch_types=[
          pltpu.VMEM_SHARED((N_PAD, FW), jnp.float32),
          pltpu.VMEM((n_chunks, CHUNK), jnp.int32),
          pltpu.VMEM((CHUNK, FW), jnp.float32),
      ])


_BR = 632
_GRID = (N_PAD // _BR,)


def _rows(width):
  return pl.BlockSpec((_BR, width), lambda i: (i, 0))


def _full(r, cdim):
  return pl.BlockSpec((r, cdim), lambda i: (0, 0))


def _mm2_body(x_ref, wl_ref, wr_ref, xl_ref, xr_ref):
  xb = x_ref[...]
  xl_ref[...] = jnp.dot(xb, wl_ref[...], preferred_element_type=jnp.float32)
  xr_ref[...] = jnp.dot(xb, wr_ref[...], preferred_element_type=jnp.float32)


def _reduce_cnt_body(c0_ref, c1_ref, cnt_ref):
  cnt_ref[...] = (c0_ref[...] + c1_ref[...])[:, 0:1]


def _combine1_body(s0_ref, s1_ref, c_ref, xr_ref, b_ref, h_ref):
  cnt = jnp.maximum(c_ref[...], 1.0)
  mean = (s0_ref[...] + s1_ref[...]) / cnt
  h_ref[...] = jnp.maximum(mean + b_ref[...] + xr_ref[...], 0.0)


def _combine2_body(s0_ref, s1_ref, c_ref, h_ref, wl_ref, wr_ref,
                   b_ref, z_ref):
  cnt = jnp.maximum(c_ref[...], 1.0)
  mean = (s0_ref[...] + s1_ref[...]) / cnt
  z_ref[...] = (jnp.dot(mean, wl_ref[...], preferred_element_type=jnp.float32)
                + jnp.dot(h_ref[...], wr_ref[...],
                          preferred_element_type=jnp.float32)
                + b_ref[...])


@jax.jit
def kernel(x, edge_index, W1l, b1l, W1r, W2l, b2l, W2r):
  n, din = x.shape
  h_dim = W1l.shape[0]
  dout = W2l.shape[0]
  e = edge_index.shape[1]

  x_pad = jnp.pad(x, ((0, N_PAD - n), (0, 0)))
  per_tile = -(-e // (NW * CHUNK)) * CHUNK
  n_chunks = per_tile // CHUNK
  if n_chunks % 2:
    n_chunks += 1
    per_tile += CHUNK
  e_pad = NW * per_tile
  dst3 = jnp.concatenate(
      [edge_index[1], jnp.full((e_pad - e,), n, jnp.int32)]).reshape(
          NW, n_chunks, CHUNK)
  nc_max = max(NB_CORE0, NB_CORE1)
  e0 = NS * NB_CORE0 * CHUNK
  e1cap = NS * NB_CORE1 * CHUNK
  npad2 = e0 + e1cap - e

  def _slabs(a, fill):
    a = jnp.concatenate([a, jnp.full((npad2,), fill, jnp.int32)])
    a0 = jnp.pad(a[:e0].reshape(NS, NB_CORE0, CHUNK),
                 ((0, 0), (0, nc_max - NB_CORE0), (0, 0)))
    a1 = jnp.pad(a[e0:].reshape(NS, NB_CORE1, CHUNK),
                 ((0, 0), (0, nc_max - NB_CORE1), (0, 0)))
    return jnp.concatenate([a0, a1], axis=0)

  src3a = _slabs(edge_index[0], 0)
  dst3a = _slabs(edge_index[1], n)
  zrow = jnp.zeros((ROWS_PER_TILE, FW), jnp.float32)
  ones = jnp.ones((CHUNK, FW), jnp.float32)

  xl, xr = pl.pallas_call(
      _mm2_body,
      grid=_GRID,
      in_specs=[_rows(din), _full(din, h_dim), _full(din, h_dim)],
      out_specs=[_rows(h_dim), _rows(h_dim)],
      out_shape=[jax.ShapeDtypeStruct((N_PAD, h_dim), jnp.float32)] * 2,
  )(x_pad, W1l.T, W1r.T)

  (cnts,) = _make_sc_degree(n_chunks)(dst3, zrow, ones)
  (sums1,) = _make_sc_aggregate(NB_CORE0, NB_CORE1, nc_max)(
      xl, src3a, dst3a, zrow)
  cn = pl.pallas_call(
      _reduce_cnt_body,
      grid=_GRID,
      in_specs=[_rows(FW), _rows(FW)],
      out_specs=pl.BlockSpec((_BR, 1), lambda i: (i, 0)),
      out_shape=jax.ShapeDtypeStruct((N_PAD, 1), jnp.float32),
  )(cnts[0], cnts[1])
  h = pl.pallas_call(
      _combine1_body,
      grid=_GRID,
      in_specs=[_rows(h_dim), _rows(h_dim),
                pl.BlockSpec((_BR, 1), lambda i: (i, 0)),
                _rows(h_dim), _full(1, h_dim)],
      out_specs=_rows(h_dim),
      out_shape=jax.ShapeDtypeStruct((N_PAD, h_dim), jnp.float32),
  )(sums1[0], sums1[1], cn, xr, b1l.reshape(1, h_dim))

  (sums2,) = _make_sc_aggregate(NB_CORE0, NB_CORE1, nc_max)(
      h, src3a, dst3a, zrow)

  z = pl.pallas_call(
      _combine2_body,
      grid=_GRID,
      in_specs=[_rows(h_dim), _rows(h_dim),
                pl.BlockSpec((_BR, 1), lambda i: (i, 0)),
                _rows(h_dim), _full(h_dim, dout), _full(h_dim, dout),
                _full(1, dout)],
      out_specs=_rows(dout),
      out_shape=jax.ShapeDtypeStruct((N_PAD, dout), jnp.float32),
  )(sums2[0], sums2[1], cn, h, W2l.T, W2r.T,
    b2l.reshape(1, dout))

  return z[:n]


# speedup vs baseline: 1.1889x; 1.1889x over previous
import functools

import jax
import jax.numpy as jnp
from jax import lax
from jax.experimental import pallas as pl
from jax.experimental.pallas import tpu as pltpu
from jax.experimental.pallas import tpu_sc as plsc

NC = 2
NS = 16
NW = NC * NS

N_PAD = 10112
ROWS_PER_TILE = N_PAD // NS
CHUNK = 128
NB_CORE0 = 60
NB_CORE1 = 20
FW = 128
CW = 16


def _make_sc_aggregate(nb0, nb1, nc_max):
  mesh = plsc.VectorSubcoreMesh(core_axis_name="c", subcore_axis_name="s")

  def body(table, src3, dst3, zrow, sums_out,
           acc, src_v, dst_v, buf, gs0, gs1, ss0, ss1):
    c = lax.axis_index("c")
    s = lax.axis_index("s")
    wid = c * NS + s
    nb = jnp.where(c == 0, nb0, nb1)
    base = s * ROWS_PER_TILE
    pltpu.sync_copy(src3.at[wid], src_v)
    pltpu.sync_copy(dst3.at[wid], dst_v)
    pltpu.sync_copy(zrow, acc.at[pl.ds(base, ROWS_PER_TILE)])
    plsc.subcore_barrier()

    gsem = (gs0, gs1)
    ssem = (ss0, ss1)

    def start_gather(j, b):
      return pltpu.async_copy(table.at[src_v.at[j]], buf.at[b], gsem[b])

    def wait_gather(b):
      pltpu.make_async_copy(table.at[src_v.at[0]], buf.at[b],
                            gsem[b]).wait()

    def start_scatter(j, b):
      return pltpu.async_copy(buf.at[b], acc.at[dst_v.at[j]], ssem[b],
                              add=True)

    def wait_scatter(b):
      pltpu.make_async_copy(buf.at[b], acc.at[dst_v.at[0]], ssem[b]).wait()

    start_gather(0, 0)
    start_gather(1, 1)

    def loop(i, carry):
      g = 2 * i
      wait_gather(0)
      start_scatter(g, 0)
      wait_gather(1)
      start_scatter(g + 1, 1)
      wait_scatter(0)
      start_gather(g + 2, 0)
      wait_scatter(1)
      start_gather(g + 3, 1)
      return carry

    lax.fori_loop(0, nb // 2 - 1, loop, 0)
    wait_gather(0)
    d0 = start_scatter(nb - 2, 0)
    wait_gather(1)
    d1 = start_scatter(nb - 1, 1)
    d0.wait()
    d1.wait()
    plsc.subcore_barrier()
    pltpu.sync_copy(acc.at[pl.ds(base, ROWS_PER_TILE)],
                    sums_out.at[c, pl.ds(base, ROWS_PER_TILE)])

  return pl.kernel(
      body,
      out_type=[jax.ShapeDtypeStruct((NC, N_PAD, FW), jnp.float32)],
      mesh=mesh,
      scratch_types=[
          pltpu.VMEM_SHARED((N_PAD, FW), jnp.float32),
          pltpu.VMEM((nc_max, CHUNK), jnp.int32),
          pltpu.VMEM((nc_max, CHUNK), jnp.int32),
          pltpu.VMEM((2, CHUNK, FW), jnp.float32),
          pltpu.SemaphoreType.DMA,
          pltpu.SemaphoreType.DMA,
          pltpu.SemaphoreType.DMA,
          pltpu.SemaphoreType.DMA,
      ])


CROWS = N_PAD // 128


def _make_sc_degree(n_chunks):
  mesh = plsc.VectorSubcoreMesh(core_axis_name="c", subcore_axis_name="s")

  def body(dst3, zrow, ones, cnts_out, cacc, dst_v, ones_v):
    c = lax.axis_index("c")
    s = lax.axis_index("s")
    wid = s * NC + c
    base = s * ROWS_PER_TILE
    pltpu.sync_copy(dst3.at[wid], dst_v)
    pltpu.sync_copy(ones, ones_v)
    pltpu.sync_copy(zrow, cacc.at[pl.ds(base, ROWS_PER_TILE)])
    plsc.subcore_barrier()

    def loop(j, carry):
      pltpu.sync_copy(ones_v, cacc.at[dst_v.at[j]], add=True)
      return carry

    lax.fori_loop(0, n_chunks, loop, 0)
    plsc.subcore_barrier()
    pltpu.sync_copy(cacc.at[pl.ds(base, ROWS_PER_TILE)],
                    cnts_out.at[c, pl.ds(base, ROWS_PER_TILE)])

  return pl.kernel(
      body,
      out_type=[jax.ShapeDtypeStruct((NC, N_PAD, FW), jnp.float32)],
      mesh=mesh,
      scratch_types=[
          pltpu.VMEM_SHARED((N_PAD, FW), jnp.float32),
          pltpu.VMEM((n_chunks, CHUNK), jnp.int32),
          pltpu.VMEM((CHUNK, FW), jnp.float32),
      ])


_BR = 632
_GRID = (N_PAD // _BR,)


def _rows(width):
  return pl.BlockSpec((_BR, width), lambda i: (i, 0))


def _full(r, cdim):
  return pl.BlockSpec((r, cdim), lambda i: (0, 0))


def _mm2_body(x_ref, wl_ref, wr_ref, xl_ref, xr_ref):
  xb = x_ref[...]
  xl_ref[...] = jnp.dot(xb, wl_ref[...], preferred_element_type=jnp.float32)
  xr_ref[...] = jnp.dot(xb, wr_ref[...], preferred_element_type=jnp.float32)


def _reduce_cnt_body(c0_ref, c1_ref, cnt_ref):
  cnt_ref[...] = (c0_ref[...] + c1_ref[...])[:, 0:1]


def _combine1_body(s0_ref, s1_ref, c_ref, xr_ref, b_ref, h_ref):
  cnt = jnp.maximum(c_ref[...], 1.0)
  mean = (s0_ref[...] + s1_ref[...]) / cnt
  h_ref[...] = jnp.maximum(mean + b_ref[...] + xr_ref[...], 0.0)


def _combine2_body(s0_ref, s1_ref, c_ref, h_ref, wl_ref, wr_ref,
                   b_ref, z_ref):
  cnt = jnp.maximum(c_ref[...], 1.0)
  mean = (s0_ref[...] + s1_ref[...]) / cnt
  z_ref[...] = (jnp.dot(mean, wl_ref[...], preferred_element_type=jnp.float32)
                + jnp.dot(h_ref[...], wr_ref[...],
                          preferred_element_type=jnp.float32)
                + b_ref[...])


@jax.jit
def kernel(x, edge_index, W1l, b1l, W1r, W2l, b2l, W2r):
  n, din = x.shape
  h_dim = W1l.shape[0]
  dout = W2l.shape[0]
  e = edge_index.shape[1]

  x_pad = jnp.pad(x, ((0, N_PAD - n), (0, 0)))
  per_tile = -(-e // (NW * CHUNK)) * CHUNK
  n_chunks = per_tile // CHUNK
  if n_chunks % 2:
    n_chunks += 1
    per_tile += CHUNK
  e_pad = NW * per_tile
  dst3 = jnp.concatenate(
      [edge_index[1], jnp.full((e_pad - e,), n, jnp.int32)]).reshape(
          NW, n_chunks, CHUNK)
  nc_max = max(NB_CORE0, NB_CORE1)
  e0 = NS * NB_CORE0 * CHUNK
  e1cap = NS * NB_CORE1 * CHUNK
  npad2 = e0 + e1cap - e

  def _slabs(a, fill):
    a = jnp.concatenate([a, jnp.full((npad2,), fill, jnp.int32)])
    a0 = jnp.pad(a[:e0].reshape(NS, NB_CORE0, CHUNK),
                 ((0, 0), (0, nc_max - NB_CORE0), (0, 0)))
    a1 = jnp.pad(a[e0:].reshape(NS, NB_CORE1, CHUNK),
                 ((0, 0), (0, nc_max - NB_CORE1), (0, 0)))
    return jnp.concatenate([a0, a1], axis=0)

  src3a = _slabs(edge_index[0], 0)
  dst3a = _slabs(edge_index[1], n)
  zrow = jnp.zeros((ROWS_PER_TILE, FW), jnp.float32)
  ones = jnp.ones((CHUNK, FW), jnp.float32)

  xl, xr = pl.pallas_call(
      _mm2_body,
      grid=_GRID,
      in_specs=[_rows(din), _full(din, h_dim), _full(din, h_dim)],
      out_specs=[_rows(h_dim), _rows(h_dim)],
      out_shape=[jax.ShapeDtypeStruct((N_PAD, h_dim), jnp.float32)] * 2,
  )(x_pad, W1l.T, W1r.T)

  (cnts,) = _make_sc_degree(n_chunks)(dst3, zrow, ones)
  (sums1,) = _make_sc_aggregate(NB_CORE0, NB_CORE1, nc_max)(
      xl, src3a, dst3a, zrow)
  cn = pl.pallas_call(
      _reduce_cnt_body,
      grid=_GRID,
      in_specs=[_rows(FW), _rows(FW)],
      out_specs=pl.BlockSpec((_BR, 1), lambda i: (i, 0)),
      out_shape=jax.ShapeDtypeStruct((N_PAD, 1), jnp.float32),
  )(cnts[0], cnts[1])
  h = pl.pallas_call(
      _combine1_body,
      grid=_GRID,
      in_specs=[_rows(h_dim), _rows(h_dim),
                pl.BlockSpec((_BR, 1), lambda i: (i, 0)),
                _rows(h_dim), _full(1, h_dim)],
      out_specs=_rows(h_dim),
      out_shape=jax.ShapeDtypeStruct((N_PAD, h_dim), jnp.float32),
  )(sums1[0], sums1[1], cn, xr, b1l.reshape(1, h_dim))

  (sums2,) = _make_sc_aggregate(NB_CORE0, NB_CORE1, nc_max)(
      h, src3a, dst3a, zrow)

  z = pl.pallas_call(
      _combine2_body,
      grid=_GRID,
      in_specs=[_rows(h_dim), _rows(h_dim),
                pl.BlockSpec((_BR, 1), lambda i: (i, 0)),
                _rows(h_dim), _full(h_dim, dout), _full(h_dim, dout),
                _full(1, dout)],
      out_specs=_rows(dout),
      out_shape=jax.ShapeDtypeStruct((N_PAD, dout), jnp.float32),
  )(sums2[0], sums2[1], cn, h, W2l.T, W2r.T,
    b2l.reshape(1, dout))

  return z[:n]
